# baseline (speedup 1.0000x reference)
import numpy as np
import jax
import jax.numpy as jnp
from jax.experimental import pallas as pl

_INTERPRET = False

_N = 2048
_TILE = 256
_K = 9

_SORT_NET = (
    (0, 3), (1, 7), (2, 5), (4, 8),
    (0, 7), (2, 4), (3, 8), (5, 6),
    (0, 2), (1, 3), (4, 5), (7, 8),
    (1, 4), (3, 6), (5, 7),
    (0, 1), (2, 4), (3, 5), (6, 8),
    (2, 3), (4, 5), (6, 7),
    (1, 2), (3, 4), (5, 6),
)

_TWO_PI = np.float32(2.0 * np.pi)


def _geom_kernel(xt_ref, xa9_ref, o_ref):
    xt = xt_ref[0]
    xa9f = xa9_ref[0]
    xa = (xa9f[0:3, :] + xa9f[3:6, :]) + xa9f[6:9, :]
    xa9 = xa9f.astype(jnp.bfloat16)

    dot = jax.lax.dot_general(
        xt, xa, (((0,), (0,)), ((), ())),
        preferred_element_type=jnp.float32)
    xx_t = jnp.sum(xt * xt, axis=0)[:, None]
    xx_a = jnp.sum(xa * xa, axis=0)[None, :]
    d = ((-xx_t) - (-2.0 * dot)) - xx_a

    cols = jax.lax.broadcasted_iota(jnp.int32, (_TILE, _N), 1)
    neg_inf = jnp.float32(-jnp.inf)

    gx, gy, gz = [], [], []
    for j in range(_K + 1):
        m = jnp.max(d, axis=1, keepdims=True)
        cand = jnp.where(d == m, cols, _N)
        idx = jnp.min(cand, axis=1, keepdims=True)
        oh_b = cols == idx
        d = jnp.where(oh_b, neg_inf, d)
        if j == 0:
            continue
        oh = oh_b.astype(jnp.bfloat16)
        dn = (((1,), (1,)), ((), ()))
        g = (jax.lax.dot_general(xa9[0:3, :], oh, dn,
                                 preferred_element_type=jnp.float32)
             + jax.lax.dot_general(xa9[3:6, :], oh, dn,
                                   preferred_element_type=jnp.float32)
             + jax.lax.dot_general(xa9[6:9, :], oh, dn,
                                   preferred_element_type=jnp.float32))
        gx.append(g[0:1, :])
        gy.append(g[1:2, :])
        gz.append(g[2:3, :])

    xq = xt[0:1, :]
    yq = xt[1:2, :]
    zq = xt[2:3, :]
    rx = [g - xq for g in gx]
    ry = [g - yq for g in gy]
    rz = [g - zq for g in gz]

    sx = rx[0]
    sy = ry[0]
    sz = rz[0]
    for j in range(1, _K):
        sx = sx + rx[j]
        sy = sy + ry[j]
        sz = sz + rz[j]
    inv_k = jnp.float32(1.0 / _K)
    mean_x = sx * inv_k
    mean_y = sy * inv_k
    mean_z = sz * inv_k

    phi = [jnp.arctan2(ry[j], rx[j]) / _TWO_PI + 0.5 for j in range(_K)]
    tid = [jnp.full((1, _TILE), np.float32(j), jnp.float32) for j in range(_K)]
    for a, b in _SORT_NET:
        pa, pb = phi[a], phi[b]
        swap = (pa > pb) | ((pa == pb) & (tid[a] > tid[b]))
        phi[a], phi[b] = jnp.where(swap, pb, pa), jnp.where(swap, pa, pb)
        tid[a], tid[b] = (jnp.where(swap, tid[b], tid[a]),
                          jnp.where(swap, tid[a], tid[b]))
        rx[a], rx[b] = jnp.where(swap, rx[b], rx[a]), jnp.where(swap, rx[a], rx[b])
        ry[a], ry[b] = jnp.where(swap, ry[b], ry[a]), jnp.where(swap, ry[a], ry[b])
        rz[a], rz[b] = jnp.where(swap, rz[b], rz[a]), jnp.where(swap, rz[a], rz[b])

    nx_acc = jnp.zeros((1, _TILE), jnp.float32)
    ny_acc = jnp.zeros((1, _TILE), jnp.float32)
    nz_acc = jnp.zeros((1, _TILE), jnp.float32)
    eps = jnp.float32(1e-6)
    one = jnp.float32(1.0)
    for j in range(_K):
        jn = (j + 1) % _K
        x1, y1, z1 = rx[j], ry[j], rz[j]
        x2, y2, z2 = rx[jn], ry[jn], rz[jn]
        cx = y1 * z2 - z1 * y2
        cy = z1 * x2 - x1 * z2
        cz = x1 * y2 - y1 * x2
        norm = jnp.sqrt(cx * cx + cy * cy + cz * cz)
        scale = one / (norm + eps)
        mz = (z1 + z2) * 0.5
        sgn = jnp.where(mz > 0.0, one, -one)
        w = sgn * scale
        nx_acc = nx_acc + cx * w
        ny_acc = ny_acc + cy * w
        nz_acc = nz_acc + cz * w

    out = jnp.concatenate(
        [mean_x, mean_y, mean_z,
         nx_acc * inv_k, ny_acc * inv_k, nz_acc * inv_k], axis=0)
    o_ref[0] = out


def kernel(x, k):
    B, N, C = x.shape
    xt = jnp.transpose(x, (0, 2, 1))
    xt_hi = jax.lax.reduce_precision(xt, exponent_bits=8, mantissa_bits=7)
    r1 = xt - xt_hi
    xt_md = jax.lax.reduce_precision(r1, exponent_bits=8, mantissa_bits=7)
    xt_lo = r1 - xt_md
    xa9 = jnp.concatenate([xt_hi, xt_md, xt_lo], axis=1)
    out_t = pl.pallas_call(
        _geom_kernel,
        grid=(B, N // _TILE),
        in_specs=[
            pl.BlockSpec((1, C, _TILE), lambda b, i: (b, 0, i)),
            pl.BlockSpec((1, 3 * C, N), lambda b, i: (b, 0, 0)),
        ],
        out_specs=pl.BlockSpec((1, 6, _TILE), lambda b, i: (b, 0, i)),
        out_shape=jax.ShapeDtypeStruct((B, 6, N), jnp.float32),
        interpret=_INTERPRET,
    )(xt, xa9)
    return jnp.transpose(out_t, (0, 2, 1))


# speedup vs baseline: 1.0484x; 1.0484x over previous
import numpy as np
import jax
import jax.numpy as jnp
from jax.experimental import pallas as pl

_INTERPRET = False

_N = 2048
_TILE = 256
_K = 9

_SORT_NET = (
    (0, 3), (1, 7), (2, 5), (4, 8),
    (0, 7), (2, 4), (3, 8), (5, 6),
    (0, 2), (1, 3), (4, 5), (7, 8),
    (1, 4), (3, 6), (5, 7),
    (0, 1), (2, 4), (3, 5), (6, 8),
    (2, 3), (4, 5), (6, 7),
    (1, 2), (3, 4), (5, 6),
)

_TWO_PI = np.float32(2.0 * np.pi)


def _geom_kernel(xt_ref, xa_ref, o_ref):
    xt = xt_ref[0]
    xa = xa_ref[0]

    dot = jax.lax.dot_general(
        xt, xa, (((0,), (0,)), ((), ())),
        preferred_element_type=jnp.float32)
    xx_t = jnp.sum(xt * xt, axis=0)[:, None]
    xx_a = jnp.sum(xa * xa, axis=0)[None, :]
    d = ((-xx_t) - (-2.0 * dot)) - xx_a

    colsf = jax.lax.broadcasted_iota(
        jnp.int32, (_TILE, _N), 1).astype(jnp.float32)
    neg_inf = jnp.float32(-jnp.inf)
    big = jnp.float32(_N)

    xa_hi = xa.astype(jnp.bfloat16).astype(jnp.float32)
    r1 = xa - xa_hi
    xa_md = r1.astype(jnp.bfloat16).astype(jnp.float32)
    xa_lo = (r1 - xa_md).astype(jnp.bfloat16)
    xa_hi = xa_hi.astype(jnp.bfloat16)
    xa_md = xa_md.astype(jnp.bfloat16)

    gx, gy, gz = [], [], []
    for j in range(_K + 1):
        m = jnp.max(d, axis=1, keepdims=True)
        candf = jnp.where(d == m, colsf, big)
        idxf = jnp.min(candf, axis=1, keepdims=True)
        oh_b = candf == idxf
        d = jnp.where(oh_b, neg_inf, d)
        if j == 0:
            continue
        oh = oh_b.astype(jnp.bfloat16)
        dn = (((1,), (1,)), ((), ()))
        g = (jax.lax.dot_general(xa_hi, oh, dn,
                                 preferred_element_type=jnp.float32)
             + jax.lax.dot_general(xa_md, oh, dn,
                                   preferred_element_type=jnp.float32)
             + jax.lax.dot_general(xa_lo, oh, dn,
                                   preferred_element_type=jnp.float32))
        gx.append(g[0:1, :])
        gy.append(g[1:2, :])
        gz.append(g[2:3, :])

    xq = xt[0:1, :]
    yq = xt[1:2, :]
    zq = xt[2:3, :]
    rx = [g - xq for g in gx]
    ry = [g - yq for g in gy]
    rz = [g - zq for g in gz]

    sx = rx[0]
    sy = ry[0]
    sz = rz[0]
    for j in range(1, _K):
        sx = sx + rx[j]
        sy = sy + ry[j]
        sz = sz + rz[j]
    inv_k = jnp.float32(1.0 / _K)
    mean_x = sx * inv_k
    mean_y = sy * inv_k
    mean_z = sz * inv_k

    phi = [jnp.arctan2(ry[j], rx[j]) / _TWO_PI + 0.5 for j in range(_K)]
    tid = [jnp.full((1, _TILE), np.float32(j), jnp.float32) for j in range(_K)]
    for a, b in _SORT_NET:
        pa, pb = phi[a], phi[b]
        swap = (pa > pb) | ((pa == pb) & (tid[a] > tid[b]))
        phi[a], phi[b] = jnp.where(swap, pb, pa), jnp.where(swap, pa, pb)
        tid[a], tid[b] = (jnp.where(swap, tid[b], tid[a]),
                          jnp.where(swap, tid[a], tid[b]))
        rx[a], rx[b] = jnp.where(swap, rx[b], rx[a]), jnp.where(swap, rx[a], rx[b])
        ry[a], ry[b] = jnp.where(swap, ry[b], ry[a]), jnp.where(swap, ry[a], ry[b])
        rz[a], rz[b] = jnp.where(swap, rz[b], rz[a]), jnp.where(swap, rz[a], rz[b])

    nx_acc = jnp.zeros((1, _TILE), jnp.float32)
    ny_acc = jnp.zeros((1, _TILE), jnp.float32)
    nz_acc = jnp.zeros((1, _TILE), jnp.float32)
    eps = jnp.float32(1e-6)
    one = jnp.float32(1.0)
    for j in range(_K):
        jn = (j + 1) % _K
        x1, y1, z1 = rx[j], ry[j], rz[j]
        x2, y2, z2 = rx[jn], ry[jn], rz[jn]
        cx = y1 * z2 - z1 * y2
        cy = z1 * x2 - x1 * z2
        cz = x1 * y2 - y1 * x2
        norm = jnp.sqrt(cx * cx + cy * cy + cz * cz)
        scale = one / (norm + eps)
        mz = (z1 + z2) * 0.5
        sgn = jnp.where(mz > 0.0, one, -one)
        w = sgn * scale
        nx_acc = nx_acc + cx * w
        ny_acc = ny_acc + cy * w
        nz_acc = nz_acc + cz * w

    out = jnp.concatenate(
        [mean_x, mean_y, mean_z,
         nx_acc * inv_k, ny_acc * inv_k, nz_acc * inv_k], axis=0)
    o_ref[0] = out


def kernel(x, k):
    B, N, C = x.shape
    xt = jnp.transpose(x, (0, 2, 1))
    out_t = pl.pallas_call(
        _geom_kernel,
        grid=(B, N // _TILE),
        in_specs=[
            pl.BlockSpec((1, C, _TILE), lambda b, i: (b, 0, i)),
            pl.BlockSpec((1, C, N), lambda b, i: (b, 0, 0)),
        ],
        out_specs=pl.BlockSpec((1, 6, _TILE), lambda b, i: (b, 0, i)),
        out_shape=jax.ShapeDtypeStruct((B, 6, N), jnp.float32),
        interpret=_INTERPRET,
    )(xt, xt)
    return jnp.transpose(out_t, (0, 2, 1))
